# Initial kernel scaffold; baseline (speedup 1.0000x reference)
import jax
import jax.numpy as jnp
from jax.experimental import pallas as pl


def kernel(data, adj_indices, adj_values, W):
    raise NotImplementedError("write your pallas kernel here")


# speedup vs baseline: 5.0705x; 5.0705x over previous
import functools

import jax
import jax.numpy as jnp
from jax import lax
from jax.experimental import pallas as pl
from jax.experimental.pallas import tpu as pltpu
from jax.experimental.pallas import tpu_sc as plsc

N = 10000
E = 320000
D = 128
H = 128
ALPHA = 0.01

NPAD = 10240
CHUNK = 128
NUM_CHUNKS = E // CHUNK
NCORES = 2
NSUB = 16
NW = NCORES * NSUB
SLABS_PER_SUB = NPAD // (CHUNK * NSUB)


def _mm_body(data_ref, noise_ref, w_ref, x_ref):
    feat = data_ref[...] + ALPHA * noise_ref[...]
    x_ref[...] = jnp.dot(feat, w_ref[...], preferred_element_type=jnp.float32)


def _project(data, noise, W):
    blk = 1000
    return pl.pallas_call(
        _mm_body,
        grid=(N // blk,),
        in_specs=[
            pl.BlockSpec((blk, D), lambda i: (i, 0)),
            pl.BlockSpec((blk, D), lambda i: (i, 0)),
            pl.BlockSpec((D, H), lambda i: (0, 0)),
        ],
        out_specs=pl.BlockSpec((blk, H), lambda i: (i, 0)),
        out_shape=jax.ShapeDtypeStruct((N, H), jnp.float32),
    )(data, noise, W)


def _spmm_body(x_hbm, row_hbm, col_hbm, val_hbm, out_hbm,
               col_v, row_v, val_v, rows_v, acc_sh, sem):
    cid = lax.axis_index("c")
    sid = lax.axis_index("s")
    wid = sid * NCORES + cid

    zeros16 = jnp.zeros((16,), jnp.float32)

    def _zrow(j, _):
        for q in range(H // 16):
            rows_v[j, pl.ds(q * 16, 16)] = zeros16
        return 0

    lax.fori_loop(0, CHUNK, _zrow, 0)
    for t in range(SLABS_PER_SUB):
        slab = (t * NSUB) * CHUNK
        pltpu.sync_copy(rows_v, acc_sh.at[pl.ds(slab + sid * CHUNK, CHUNK)])
    plsc.subcore_barrier()

    n_iters = (NUM_CHUNKS - wid + NW - 1) // NW

    def _chunk(i, _):
        base = (wid + i * NW) * CHUNK
        pltpu.sync_copy(col_hbm.at[pl.ds(base, CHUNK)], col_v)
        pltpu.sync_copy(row_hbm.at[pl.ds(base, CHUNK)], row_v)
        pltpu.sync_copy(val_hbm.at[pl.ds(base, CHUNK)], val_v)
        pltpu.async_copy(x_hbm.at[col_v], rows_v, sem).wait()

        def _scale(g, _):
            vv = val_v[pl.ds(g * 16, 16)]
            for lane in range(16):
                v = vv[lane]
                j = g * 16 + lane
                for q in range(H // 16):
                    rows_v[j, pl.ds(q * 16, 16)] = rows_v[j, pl.ds(q * 16, 16)] * v
            return 0

        lax.fori_loop(0, CHUNK // 16, _scale, 0)
        pltpu.sync_copy(rows_v, acc_sh.at[row_v], add=True)
        return 0

    lax.fori_loop(0, n_iters, _chunk, 0)
    plsc.subcore_barrier()

    for t in range(SLABS_PER_SUB):
        slab = (t * NSUB + 0) * CHUNK + sid * CHUNK
        pltpu.sync_copy(acc_sh.at[pl.ds(slab, CHUNK)],
                        out_hbm.at[cid, pl.ds(slab, CHUNK)])


def _spmm_partials(x, row, col, vals):
    mesh = plsc.VectorSubcoreMesh(core_axis_name="c", subcore_axis_name="s")
    f = pl.kernel(
        _spmm_body,
        out_type=jax.ShapeDtypeStruct((NCORES, NPAD, H), jnp.float32),
        mesh=mesh,
        scratch_types=[
            pltpu.VMEM((CHUNK,), jnp.int32),
            pltpu.VMEM((CHUNK,), jnp.int32),
            pltpu.VMEM((CHUNK,), jnp.float32),
            pltpu.VMEM((CHUNK, H), jnp.float32),
            pltpu.VMEM_SHARED((NPAD, H), jnp.float32),
            pltpu.SemaphoreType.DMA,
        ],
    )
    return f(x, row, col, vals)


def _fin_body(p_ref, out_ref):
    s = p_ref[0] + p_ref[1]
    out_ref[...] = jnp.where(s > 0, s, jnp.exp(s) - 1.0)


def _finish(partials):
    blk = 1000
    return pl.pallas_call(
        _fin_body,
        grid=(N // blk,),
        in_specs=[pl.BlockSpec((NCORES, blk, H), lambda i: (0, i, 0))],
        out_specs=pl.BlockSpec((blk, H), lambda i: (i, 0)),
        out_shape=jax.ShapeDtypeStruct((N, H), jnp.float32),
    )(partials)


def kernel(data, adj_indices, adj_values, W):
    noise = jax.random.normal(jax.random.key(42), data.shape, dtype=data.dtype)
    x = _project(data, noise, W)
    row = adj_indices[0]
    col = adj_indices[1]
    partials = _spmm_partials(x, row, col, adj_values)
    return _finish(partials)
